# Initial kernel scaffold; baseline (speedup 1.0000x reference)
import jax
import jax.numpy as jnp
from jax.experimental import pallas as pl


def kernel(x, edge_index, edge_attr, W1, b1, W2, b2, Wlin, blin):
    raise NotImplementedError("write your pallas kernel here")


# speedup vs baseline: 36.2635x; 36.2635x over previous
import functools

import jax
import jax.numpy as jnp
from jax import lax
from jax.experimental import pallas as pl
from jax.experimental.pallas import tpu as pltpu
from jax.experimental.pallas import tpu_sc as plsc

N = 100000
E = 3200000
F = 16

NC = 2
NS = 16
NW = NC * NS
PT = 6272
N2 = NS * PT
EPT = 100352
E2 = NW * EPT
ROWS_PT = EPT // 128
NOUT = 98
BLK = 6272
GRID = N2 // BLK

_mesh = plsc.VectorSubcoreMesh(core_axis_name="c", subcore_axis_name="s")


@functools.partial(
    pl.kernel,
    out_type=(
        jax.ShapeDtypeStruct((N2,), jnp.float32),
        jax.ShapeDtypeStruct((N2,), jnp.float32),
    ),
    mesh=_mesh,
    scratch_types=[
        pltpu.VMEM_SHARED((N2,), jnp.float32),
        pltpu.VMEM((8, 128), jnp.int32),
        pltpu.VMEM((8, 128), jnp.float32),
        pltpu.VMEM((1024,), jnp.float32),
    ],
)
def _deg_kernel(dst_hbm, w_hbm, out0, out1, deg_sh, didx, wb, zbuf):
    cid = lax.axis_index("c")
    sid = lax.axis_index("s")
    tid = sid * NC + cid

    z16 = jnp.zeros((16,), jnp.float32)
    for i in range(64):
        zbuf[pl.ds(i * 16, 16)] = z16
    for r in range(6):
        pltpu.sync_copy(zbuf, deg_sh.at[pl.ds(sid * PT + r * 1024, 1024)])
    pltpu.sync_copy(zbuf.at[pl.ds(0, 128)],
                    deg_sh.at[pl.ds(sid * PT + 6144, 128)])
    plsc.subcore_barrier()

    def body(j, carry):
        rb = tid * ROWS_PT + j * 8
        pltpu.sync_copy(dst_hbm.at[pl.ds(rb, 8)], didx)
        pltpu.sync_copy(w_hbm.at[pl.ds(rb, 8)], wb)
        for q in range(8):
            pltpu.sync_copy(wb.at[q], deg_sh.at[didx.at[q]], add=True)
        return carry

    lax.fori_loop(0, NOUT, body, 0)
    plsc.subcore_barrier()

    sl = pl.ds(sid * PT, PT)

    @pl.when(cid == 0)
    def _():
        pltpu.sync_copy(deg_sh.at[sl], out0.at[sl])

    @pl.when(cid == 1)
    def _():
        pltpu.sync_copy(deg_sh.at[sl], out1.at[sl])


@functools.partial(
    pl.kernel,
    out_type=(
        jax.ShapeDtypeStruct((2, N2, F), jnp.float32),
        jax.ShapeDtypeStruct((N2,), jnp.float32),
        jax.ShapeDtypeStruct((N2,), jnp.float32),
    ),
    mesh=_mesh,
    scratch_types=[
        pltpu.VMEM_SHARED((N2, F), jnp.float32),
        pltpu.VMEM_SHARED((N2,), jnp.float32),
        pltpu.VMEM((8, 128), jnp.float32),
        pltpu.VMEM((8, 128), jnp.int32),
        pltpu.VMEM((8, 128), jnp.int32),
        pltpu.VMEM((8, 128), jnp.float32),
        pltpu.VMEM((1024, F), jnp.float32),
        pltpu.VMEM((8, 128), jnp.float32),
        pltpu.VMEM((1024,), jnp.float32),
    ],
    compiler_params=pltpu.CompilerParams(use_tc_tiling_on_sc=False),
)
def _edge_kernel(src_hbm, dst_hbm, w_hbm, y_hbm, dinv_hbm,
                 accp, c0, c1,
                 acc_sh, c_sh, dgat, sidx, didx, wb, rows, cprod,
                 zbuf):
    cid = lax.axis_index("c")
    sid = lax.axis_index("s")
    tid = sid * NC + cid

    z16 = jnp.zeros((16,), jnp.float32)
    for i in range(1024):
        rows[i, :] = z16
    for i in range(64):
        zbuf[pl.ds(i * 16, 16)] = z16
    for r in range(6):
        pltpu.sync_copy(rows, acc_sh.at[pl.ds(sid * PT + r * 1024, 1024)])
        pltpu.sync_copy(zbuf, c_sh.at[pl.ds(sid * PT + r * 1024, 1024)])
    pltpu.sync_copy(rows.at[pl.ds(0, 128)],
                    acc_sh.at[pl.ds(sid * PT + 6144, 128)])
    pltpu.sync_copy(zbuf.at[pl.ds(0, 128)],
                    c_sh.at[pl.ds(sid * PT + 6144, 128)])
    plsc.subcore_barrier()

    def body(j, carry):
        rb = tid * ROWS_PT + j * 8
        pltpu.sync_copy(src_hbm.at[pl.ds(rb, 8)], sidx)
        pltpu.sync_copy(dst_hbm.at[pl.ds(rb, 8)], didx)
        pltpu.sync_copy(w_hbm.at[pl.ds(rb, 8)], wb)
        for q in range(8):
            pltpu.sync_copy(y_hbm.at[sidx.at[q]],
                            rows.at[pl.ds(q * 128, 128)])
            pltpu.sync_copy(dinv_hbm.at[didx.at[q]], dgat.at[q])
            for g in range(8):
                w16 = wb[q, pl.ds(g * 16, 16)]
                for e in range(16):
                    k = q * 128 + g * 16 + e
                    rows[k, :] = rows[k, :] * w16[e]
                cprod[q, pl.ds(g * 16, 16)] = w16 * dgat[q, pl.ds(g * 16, 16)]
            pltpu.sync_copy(rows.at[pl.ds(q * 128, 128)],
                            acc_sh.at[didx.at[q]], add=True)
            pltpu.sync_copy(cprod.at[q], c_sh.at[sidx.at[q]], add=True)
        return carry

    lax.fori_loop(0, NOUT, body, 0)
    plsc.subcore_barrier()

    sl = pl.ds(sid * PT, PT)
    pltpu.sync_copy(acc_sh.at[sl], accp.at[cid, sl])

    @pl.when(cid == 0)
    def _():
        pltpu.sync_copy(c_sh.at[sl], c0.at[sl])

    @pl.when(cid == 1)
    def _():
        pltpu.sync_copy(c_sh.at[sl], c1.at[sl])


def _dense1_body(i_ref_unused, x_ref, w1_ref, d0_ref, d1_ref,
                 dinv_ref, y_ref):
    i = pl.program_id(0)
    deg = d0_ref[...] + d1_ref[...] + 1.0
    dv = lax.rsqrt(deg)
    rowid = lax.broadcasted_iota(jnp.int32, (BLK, 1), 0) + i * BLK
    dv = jnp.where(rowid < N, dv, 0.0)
    xw = jnp.dot(x_ref[...], w1_ref[...], preferred_element_type=jnp.float32)
    dinv_ref[...] = dv
    y_ref[...] = dv * xw


def _dense1(xp, W1, d0, d1):
    return pl.pallas_call(
        functools.partial(_dense1_body, None),
        grid=(GRID,),
        in_specs=[
            pl.BlockSpec((BLK, F), lambda i: (i, 0)),
            pl.BlockSpec((F, F), lambda i: (0, 0)),
            pl.BlockSpec((BLK, 1), lambda i: (i, 0)),
            pl.BlockSpec((BLK, 1), lambda i: (i, 0)),
        ],
        out_specs=[
            pl.BlockSpec((BLK, 1), lambda i: (i, 0)),
            pl.BlockSpec((BLK, F), lambda i: (i, 0)),
        ],
        out_shape=[
            jax.ShapeDtypeStruct((N2, 1), jnp.float32),
            jax.ShapeDtypeStruct((N2, F), jnp.float32),
        ],
    )(xp, W1, d0, d1)


def _dense2_body(accp_ref, y_ref, dinv_ref, c0_ref, c1_ref, b1_ref,
                 w2_ref, b2_ref, wlin_ref, blin_ref, out_ref, s_ref):
    i = pl.program_id(0)
    dv = dinv_ref[...]
    acc = accp_ref[0] + accp_ref[1]
    h = jax.nn.relu((acc + y_ref[...]) * dv + b1_ref[...])
    coef = dv * (c0_ref[...] + c1_ref[...] + dv)
    part = lax.dot_general(coef, h, (((0,), (0,)), ((), ())),
                           preferred_element_type=jnp.float32)

    @pl.when(i == 0)
    def _():
        s_ref[...] = part

    @pl.when(i > 0)
    def _():
        s_ref[...] = s_ref[...] + part

    @pl.when(i == GRID - 1)
    def _():
        g = jnp.dot(s_ref[...] / float(N), w2_ref[...],
                    preferred_element_type=jnp.float32) + b2_ref[...]
        out_ref[...] = jnp.dot(g, wlin_ref[...],
                               preferred_element_type=jnp.float32) \
            + blin_ref[...]


def _dense2(accp, y, dinv, c0, c1, b1, W2, b2, Wlin, blin):
    return pl.pallas_call(
        _dense2_body,
        grid=(GRID,),
        in_specs=[
            pl.BlockSpec((2, BLK, F), lambda i: (0, i, 0)),
            pl.BlockSpec((BLK, F), lambda i: (i, 0)),
            pl.BlockSpec((BLK, 1), lambda i: (i, 0)),
            pl.BlockSpec((BLK, 1), lambda i: (i, 0)),
            pl.BlockSpec((BLK, 1), lambda i: (i, 0)),
            pl.BlockSpec((1, F), lambda i: (0, 0)),
            pl.BlockSpec((F, F), lambda i: (0, 0)),
            pl.BlockSpec((1, F), lambda i: (0, 0)),
            pl.BlockSpec((F, 1), lambda i: (0, 0)),
            pl.BlockSpec((1, 1), lambda i: (0, 0)),
        ],
        out_specs=pl.BlockSpec((1, 1), lambda i: (0, 0)),
        out_shape=jax.ShapeDtypeStruct((1, 1), jnp.float32),
        scratch_shapes=[pltpu.VMEM((1, F), jnp.float32)],
    )(accp, y, dinv, c0, c1, b1, W2, b2, Wlin, blin)


def kernel(x, edge_index, edge_attr, W1, b1, W2, b2, Wlin, blin):
    src = edge_index[0]
    dst = edge_index[1]
    pad = E2 - E
    src2 = jnp.pad(src, (0, pad)).reshape(-1, 128)
    dst2 = jnp.pad(dst, (0, pad)).reshape(-1, 128)
    w2e = jnp.pad(edge_attr, (0, pad)).reshape(-1, 128)
    xp = jnp.pad(x, ((0, N2 - N), (0, 0)))

    d0, d1 = _deg_kernel(dst2, w2e)
    dinv, y = _dense1(xp, W1, d0.reshape(N2, 1), d1.reshape(N2, 1))
    accp, c0, c1 = _edge_kernel(src2, dst2, w2e, y, dinv.reshape(N2))
    out = _dense2(accp, y, dinv, c0.reshape(N2, 1), c1.reshape(N2, 1),
                  b1.reshape(1, F), W2, b2.reshape(1, F), Wlin,
                  blin.reshape(1, 1))
    return out


# speedup vs baseline: 71.8109x; 1.9802x over previous
import functools

import jax
import jax.numpy as jnp
from jax import lax
from jax.experimental import pallas as pl
from jax.experimental.pallas import tpu as pltpu
from jax.experimental.pallas import tpu_sc as plsc

N = 100000
E = 3200000
F = 16

NC = 2
NS = 16
NW = NC * NS
PT = 6272
N2 = NS * PT
EPT = 100352
E2 = NW * EPT
ROWS_PT = EPT // 128
NOUT = 98
QR = 4
NOUT_E = EPT // (QR * 128)
BLK = 6272
GRID = N2 // BLK

_mesh = plsc.VectorSubcoreMesh(core_axis_name="c", subcore_axis_name="s")


@functools.partial(
    pl.kernel,
    out_type=(
        jax.ShapeDtypeStruct((N2,), jnp.float32),
        jax.ShapeDtypeStruct((N2,), jnp.float32),
    ),
    mesh=_mesh,
    scratch_types=[
        pltpu.VMEM_SHARED((N2,), jnp.float32),
        pltpu.VMEM((2, 8, 128), jnp.int32),
        pltpu.VMEM((2, 8, 128), jnp.float32),
        pltpu.VMEM((1024,), jnp.float32),
        pltpu.SemaphoreType.DMA,
        pltpu.SemaphoreType.DMA,
    ],
)
def _deg_kernel(dst_hbm, w_hbm, out0, out1, deg_sh, didx, wb, zbuf,
                insem, ssem):
    cid = lax.axis_index("c")
    sid = lax.axis_index("s")
    tid = sid * NC + cid

    z16 = jnp.zeros((16,), jnp.float32)
    for i in range(64):
        zbuf[pl.ds(i * 16, 16)] = z16
    for r in range(6):
        pltpu.sync_copy(zbuf, deg_sh.at[pl.ds(sid * PT + r * 1024, 1024)])
    pltpu.sync_copy(zbuf.at[pl.ds(0, 128)],
                    deg_sh.at[pl.ds(sid * PT + 6144, 128)])
    plsc.subcore_barrier()

    ebase = tid * ROWS_PT

    def issue_in(j, s):
        rb = ebase + j * 8
        return [
            pltpu.async_copy(dst_hbm.at[pl.ds(rb, 8)], didx.at[s], insem),
            pltpu.async_copy(w_hbm.at[pl.ds(rb, 8)], wb.at[s], insem),
        ]

    def drain_in(j, s):
        rb = ebase + j * 8
        pltpu.make_async_copy(dst_hbm.at[pl.ds(rb, 8)], didx.at[s],
                              insem).wait()
        pltpu.make_async_copy(w_hbm.at[pl.ds(rb, 8)], wb.at[s],
                              insem).wait()

    def issue_scatters(s):
        for q in range(8):
            pltpu.async_copy(wb.at[s, q], deg_sh.at[didx.at[s, q]], ssem,
                             add=True)

    def drain_scatters(s):
        for q in range(8):
            pltpu.make_async_copy(wb.at[s, q], deg_sh.at[didx.at[s, q]],
                                  ssem).wait()

    for d in issue_in(0, 0):
        d.wait()

    def group(g, carry):
        j0 = 2 * g

        @pl.when(g > 0)
        def _():
            drain_scatters(1)
        issue_in(j0 + 1, 1)
        issue_scatters(0)
        drain_in(j0 + 1, 1)
        issue_scatters(1)
        drain_scatters(0)

        @pl.when(g < NOUT // 2 - 1)
        def _():
            issue_in(j0 + 2, 0)
            drain_in(j0 + 2, 0)
        return carry

    lax.fori_loop(0, NOUT // 2, group, 0)
    drain_scatters(1)
    plsc.subcore_barrier()

    sl = pl.ds(sid * PT, PT)

    @pl.when(cid == 0)
    def _():
        pltpu.sync_copy(deg_sh.at[sl], out0.at[sl])

    @pl.when(cid == 1)
    def _():
        pltpu.sync_copy(deg_sh.at[sl], out1.at[sl])


@functools.partial(
    pl.kernel,
    out_type=(
        jax.ShapeDtypeStruct((2, N2, F), jnp.float32),
        jax.ShapeDtypeStruct((N2,), jnp.float32),
        jax.ShapeDtypeStruct((N2,), jnp.float32),
    ),
    mesh=_mesh,
    scratch_types=[
        pltpu.VMEM_SHARED((N2, F), jnp.float32),
        pltpu.VMEM_SHARED((N2,), jnp.float32),
        pltpu.VMEM((2, QR, 128), jnp.float32),
        pltpu.VMEM((2, QR, 128), jnp.int32),
        pltpu.VMEM((2, QR, 128), jnp.int32),
        pltpu.VMEM((2, QR, 128), jnp.float32),
        pltpu.VMEM((2, QR, 128, F), jnp.float32),
        pltpu.VMEM((2, QR, 128), jnp.float32),
        pltpu.VMEM((512,), jnp.float32),
        pltpu.VMEM((64, F), jnp.float32),
        pltpu.SemaphoreType.DMA,
        pltpu.SemaphoreType.DMA,
        pltpu.SemaphoreType.DMA,
    ],
    compiler_params=pltpu.CompilerParams(use_tc_tiling_on_sc=False),
)
def _edge_kernel(src_hbm, dst_hbm, w_hbm, y_hbm, dinv_hbm,
                 accp, c0, c1,
                 acc_sh, c_sh, dgat, sidx, didx, wb, rows, cprod,
                 zbuf, zrows, insem, gsem, ssem):
    cid = lax.axis_index("c")
    sid = lax.axis_index("s")
    tid = sid * NC + cid

    z16 = jnp.zeros((16,), jnp.float32)
    for i in range(64):
        zrows[i, :] = z16
    for i in range(32):
        zbuf[pl.ds(i * 16, 16)] = z16
    for r in range(98):
        pltpu.sync_copy(zrows, acc_sh.at[pl.ds(sid * PT + r * 64, 64)])
    for r in range(12):
        pltpu.sync_copy(zbuf, c_sh.at[pl.ds(sid * PT + r * 512, 512)])
    pltpu.sync_copy(zbuf.at[pl.ds(0, 128)],
                    c_sh.at[pl.ds(sid * PT + 6144, 128)])
    plsc.subcore_barrier()

    ebase = tid * ROWS_PT

    def issue_in(j, s):
        rb = ebase + j * QR
        return [
            pltpu.async_copy(src_hbm.at[pl.ds(rb, QR)], sidx.at[s], insem),
            pltpu.async_copy(dst_hbm.at[pl.ds(rb, QR)], didx.at[s], insem),
            pltpu.async_copy(w_hbm.at[pl.ds(rb, QR)], wb.at[s], insem),
        ]

    def drain_in(j, s):
        rb = ebase + j * QR
        pltpu.make_async_copy(src_hbm.at[pl.ds(rb, QR)], sidx.at[s],
                              insem).wait()
        pltpu.make_async_copy(dst_hbm.at[pl.ds(rb, QR)], didx.at[s],
                              insem).wait()
        pltpu.make_async_copy(w_hbm.at[pl.ds(rb, QR)], wb.at[s],
                              insem).wait()

    def issue_gathers(s):
        for q in range(QR):
            pltpu.async_copy(y_hbm.at[sidx.at[s, q]], rows.at[s, q], gsem)
            pltpu.async_copy(dinv_hbm.at[didx.at[s, q]], dgat.at[s, q], gsem)

    def drain_gathers(s):
        for q in range(QR):
            pltpu.make_async_copy(y_hbm.at[sidx.at[s, q]], rows.at[s, q],
                                  gsem).wait()
            pltpu.make_async_copy(dinv_hbm.at[didx.at[s, q]], dgat.at[s, q],
                                  gsem).wait()

    def issue_scatters(s):
        for q in range(QR):
            pltpu.async_copy(rows.at[s, q], acc_sh.at[didx.at[s, q]], ssem,
                             add=True)
            pltpu.async_copy(cprod.at[s, q], c_sh.at[sidx.at[s, q]], ssem,
                             add=True)

    def drain_scatters(s):
        for q in range(QR):
            pltpu.make_async_copy(rows.at[s, q], acc_sh.at[didx.at[s, q]],
                                  ssem).wait()
            pltpu.make_async_copy(cprod.at[s, q], c_sh.at[sidx.at[s, q]],
                                  ssem).wait()

    def compute(s):
        def qbody(q, carry):
            for g in range(8):
                w16 = wb[s, q, pl.ds(g * 16, 16)]
                dv16 = dgat[s, q, pl.ds(g * 16, 16)]
                cprod[s, q, pl.ds(g * 16, 16)] = w16 * dv16
                for e in range(16):
                    k = g * 16 + e
                    rows[s, q, k, :] = rows[s, q, k, :] * w16[e]
            return carry
        lax.fori_loop(0, QR, qbody, 0)

    for d in issue_in(0, 0):
        d.wait()
    issue_gathers(0)

    def group(g, carry):
        j0 = 2 * g
        @pl.when(g > 0)
        def _():
            drain_scatters(1)
        issue_in(j0 + 1, 1)
        drain_gathers(0)
        compute(0)
        issue_scatters(0)
        drain_in(j0 + 1, 1)
        issue_gathers(1)
        drain_scatters(0)
        drain_gathers(1)
        compute(1)
        issue_scatters(1)

        @pl.when(g < NOUT_E // 2 - 1)
        def _():
            issue_in(j0 + 2, 0)
            drain_in(j0 + 2, 0)
            issue_gathers(0)
        return carry

    lax.fori_loop(0, NOUT_E // 2, group, 0)
    drain_scatters(1)
    plsc.subcore_barrier()

    sl = pl.ds(sid * PT, PT)
    pltpu.sync_copy(acc_sh.at[sl], accp.at[cid, sl])

    @pl.when(cid == 0)
    def _():
        pltpu.sync_copy(c_sh.at[sl], c0.at[sl])

    @pl.when(cid == 1)
    def _():
        pltpu.sync_copy(c_sh.at[sl], c1.at[sl])


def _dense1_body(i_ref_unused, x_ref, w1_ref, d0_ref, d1_ref,
                 dinv_ref, y_ref):
    i = pl.program_id(0)
    deg = d0_ref[...] + d1_ref[...] + 1.0
    dv = lax.rsqrt(deg)
    rowid = lax.broadcasted_iota(jnp.int32, (BLK, 1), 0) + i * BLK
    dv = jnp.where(rowid < N, dv, 0.0)
    xw = jnp.dot(x_ref[...], w1_ref[...], preferred_element_type=jnp.float32)
    dinv_ref[...] = dv
    y_ref[...] = dv * xw


def _dense1(xp, W1, d0, d1):
    return pl.pallas_call(
        functools.partial(_dense1_body, None),
        grid=(GRID,),
        in_specs=[
            pl.BlockSpec((BLK, F), lambda i: (i, 0)),
            pl.BlockSpec((F, F), lambda i: (0, 0)),
            pl.BlockSpec((BLK, 1), lambda i: (i, 0)),
            pl.BlockSpec((BLK, 1), lambda i: (i, 0)),
        ],
        out_specs=[
            pl.BlockSpec((BLK, 1), lambda i: (i, 0)),
            pl.BlockSpec((BLK, F), lambda i: (i, 0)),
        ],
        out_shape=[
            jax.ShapeDtypeStruct((N2, 1), jnp.float32),
            jax.ShapeDtypeStruct((N2, F), jnp.float32),
        ],
    )(xp, W1, d0, d1)


def _dense2_body(accp_ref, y_ref, dinv_ref, c0_ref, c1_ref, b1_ref,
                 w2_ref, b2_ref, wlin_ref, blin_ref, out_ref, s_ref):
    i = pl.program_id(0)
    dv = dinv_ref[...]
    acc = accp_ref[0] + accp_ref[1]
    h = jax.nn.relu((acc + y_ref[...]) * dv + b1_ref[...])
    coef = dv * (c0_ref[...] + c1_ref[...] + dv)
    part = lax.dot_general(coef, h, (((0,), (0,)), ((), ())),
                           preferred_element_type=jnp.float32)

    @pl.when(i == 0)
    def _():
        s_ref[...] = part

    @pl.when(i > 0)
    def _():
        s_ref[...] = s_ref[...] + part

    @pl.when(i == GRID - 1)
    def _():
        g = jnp.dot(s_ref[...] / float(N), w2_ref[...],
                    preferred_element_type=jnp.float32) + b2_ref[...]
        out_ref[...] = jnp.dot(g, wlin_ref[...],
                               preferred_element_type=jnp.float32) \
            + blin_ref[...]


def _dense2(accp, y, dinv, c0, c1, b1, W2, b2, Wlin, blin):
    return pl.pallas_call(
        _dense2_body,
        grid=(GRID,),
        in_specs=[
            pl.BlockSpec((2, BLK, F), lambda i: (0, i, 0)),
            pl.BlockSpec((BLK, F), lambda i: (i, 0)),
            pl.BlockSpec((BLK, 1), lambda i: (i, 0)),
            pl.BlockSpec((BLK, 1), lambda i: (i, 0)),
            pl.BlockSpec((BLK, 1), lambda i: (i, 0)),
            pl.BlockSpec((1, F), lambda i: (0, 0)),
            pl.BlockSpec((F, F), lambda i: (0, 0)),
            pl.BlockSpec((1, F), lambda i: (0, 0)),
            pl.BlockSpec((F, 1), lambda i: (0, 0)),
            pl.BlockSpec((1, 1), lambda i: (0, 0)),
        ],
        out_specs=pl.BlockSpec((1, 1), lambda i: (0, 0)),
        out_shape=jax.ShapeDtypeStruct((1, 1), jnp.float32),
        scratch_shapes=[pltpu.VMEM((1, F), jnp.float32)],
    )(accp, y, dinv, c0, c1, b1, W2, b2, Wlin, blin)


def kernel(x, edge_index, edge_attr, W1, b1, W2, b2, Wlin, blin):
    src = edge_index[0]
    dst = edge_index[1]
    pad = E2 - E
    src2 = jnp.pad(src, (0, pad)).reshape(-1, 128)
    dst2 = jnp.pad(dst, (0, pad)).reshape(-1, 128)
    w2e = jnp.pad(edge_attr, (0, pad)).reshape(-1, 128)
    xp = jnp.pad(x, ((0, N2 - N), (0, 0)))

    d0, d1 = _deg_kernel(dst2, w2e)
    dinv, y = _dense1(xp, W1, d0.reshape(N2, 1), d1.reshape(N2, 1))
    accp, c0, c1 = _edge_kernel(src2, dst2, w2e, y, dinv.reshape(N2))
    out = _dense2(accp, y, dinv, c0.reshape(N2, 1), c1.reshape(N2, 1),
                  b1.reshape(1, F), W2, b2.reshape(1, F), Wlin,
                  blin.reshape(1, 1))
    return out


# speedup vs baseline: 100.9569x; 1.4059x over previous
import functools

import jax
import jax.numpy as jnp
from jax import lax
from jax.experimental import pallas as pl
from jax.experimental.pallas import tpu as pltpu
from jax.experimental.pallas import tpu_sc as plsc

N = 100000
E = 3200000
F = 16

NC = 2
NS = 16
NW = NC * NS
PT = 6272
N2 = NS * PT
EPT = 100352
E2 = NW * EPT
ROWS_PT = EPT // 128
NOUT = 98
QR = 4
NOUT_E = EPT // (QR * 128)
BLK = 6272
GRID = N2 // BLK

_mesh = plsc.VectorSubcoreMesh(core_axis_name="c", subcore_axis_name="s")


@functools.partial(
    pl.kernel,
    out_type=(
        jax.ShapeDtypeStruct((N2,), jnp.float32),
        jax.ShapeDtypeStruct((N2,), jnp.float32),
    ),
    mesh=_mesh,
    scratch_types=[
        pltpu.VMEM_SHARED((N2,), jnp.float32),
        pltpu.VMEM((2, 8, 128), jnp.int32),
        pltpu.VMEM((2, 8, 128), jnp.float32),
        pltpu.VMEM((1024,), jnp.float32),
        pltpu.SemaphoreType.DMA,
        pltpu.SemaphoreType.DMA,
    ],
)
def _deg_kernel(dst_hbm, w_hbm, out0, out1, deg_sh, didx, wb, zbuf,
                insem, ssem):
    cid = lax.axis_index("c")
    sid = lax.axis_index("s")
    tid = sid * NC + cid

    z16 = jnp.zeros((16,), jnp.float32)
    for i in range(64):
        zbuf[pl.ds(i * 16, 16)] = z16
    for r in range(6):
        pltpu.sync_copy(zbuf, deg_sh.at[pl.ds(sid * PT + r * 1024, 1024)])
    pltpu.sync_copy(zbuf.at[pl.ds(0, 128)],
                    deg_sh.at[pl.ds(sid * PT + 6144, 128)])
    plsc.subcore_barrier()

    ebase = tid * ROWS_PT

    def issue_in(j, s):
        rb = ebase + j * 8
        return [
            pltpu.async_copy(dst_hbm.at[pl.ds(rb, 8)], didx.at[s], insem),
            pltpu.async_copy(w_hbm.at[pl.ds(rb, 8)], wb.at[s], insem),
        ]

    def drain_in(j, s):
        rb = ebase + j * 8
        pltpu.make_async_copy(dst_hbm.at[pl.ds(rb, 8)], didx.at[s],
                              insem).wait()
        pltpu.make_async_copy(w_hbm.at[pl.ds(rb, 8)], wb.at[s],
                              insem).wait()

    def issue_scatters(s):
        for q in range(8):
            pltpu.async_copy(wb.at[s, q], deg_sh.at[didx.at[s, q]], ssem,
                             add=True)

    def drain_scatters(s):
        for q in range(8):
            pltpu.make_async_copy(wb.at[s, q], deg_sh.at[didx.at[s, q]],
                                  ssem).wait()

    for d in issue_in(0, 0):
        d.wait()

    def group(g, carry):
        j0 = 2 * g

        @pl.when(g > 0)
        def _():
            drain_scatters(1)
        issue_in(j0 + 1, 1)
        issue_scatters(0)
        drain_in(j0 + 1, 1)
        issue_scatters(1)
        drain_scatters(0)

        @pl.when(g < NOUT // 2 - 1)
        def _():
            issue_in(j0 + 2, 0)
            drain_in(j0 + 2, 0)
        return carry

    lax.fori_loop(0, NOUT // 2, group, 0)
    drain_scatters(1)
    plsc.subcore_barrier()

    sl = pl.ds(sid * PT, PT)

    @pl.when(cid == 0)
    def _():
        pltpu.sync_copy(deg_sh.at[sl], out0.at[sl])

    @pl.when(cid == 1)
    def _():
        pltpu.sync_copy(deg_sh.at[sl], out1.at[sl])


@functools.partial(
    pl.kernel,
    out_type=(
        jax.ShapeDtypeStruct((2, N2, F), jnp.float32),
        jax.ShapeDtypeStruct((N2,), jnp.float32),
        jax.ShapeDtypeStruct((N2,), jnp.float32),
    ),
    mesh=_mesh,
    scratch_types=[
        pltpu.VMEM_SHARED((N2, F), jnp.float32),
        pltpu.VMEM_SHARED((N2,), jnp.float32),
        pltpu.VMEM((2, QR, 128), jnp.float32),
        pltpu.VMEM((2, QR, 128), jnp.int32),
        pltpu.VMEM((2, QR, 128), jnp.int32),
        pltpu.VMEM((2, QR, 128), jnp.float32),
        pltpu.VMEM((2, QR, 128, F), jnp.float32),
        pltpu.VMEM((2, QR, 128), jnp.float32),
        pltpu.VMEM((512,), jnp.float32),
        pltpu.VMEM((64, F), jnp.float32),
        pltpu.SemaphoreType.DMA,
        pltpu.SemaphoreType.DMA,
        pltpu.SemaphoreType.DMA,
    ],
    compiler_params=pltpu.CompilerParams(use_tc_tiling_on_sc=False),
)
def _edge_kernel(src_hbm, dst_hbm, w_hbm, y_hbm, dinv_hbm,
                 accp, c0, c1,
                 acc_sh, c_sh, dgat, sidx, didx, wb, rows, cprod,
                 zbuf, zrows, insem, gsem, ssem):
    cid = lax.axis_index("c")
    sid = lax.axis_index("s")
    tid = sid * NC + cid

    z16 = jnp.zeros((16,), jnp.float32)
    for i in range(64):
        zrows[i, :] = z16
    for i in range(32):
        zbuf[pl.ds(i * 16, 16)] = z16
    for r in range(98):
        pltpu.sync_copy(zrows, acc_sh.at[pl.ds(sid * PT + r * 64, 64)])
    for r in range(12):
        pltpu.sync_copy(zbuf, c_sh.at[pl.ds(sid * PT + r * 512, 512)])
    pltpu.sync_copy(zbuf.at[pl.ds(0, 128)],
                    c_sh.at[pl.ds(sid * PT + 6144, 128)])
    plsc.subcore_barrier()

    ebase = tid * ROWS_PT

    def issue_in(j, s):
        rb = ebase + j * QR
        return [
            pltpu.async_copy(src_hbm.at[pl.ds(rb, QR)], sidx.at[s], insem),
            pltpu.async_copy(dst_hbm.at[pl.ds(rb, QR)], didx.at[s], insem),
            pltpu.async_copy(w_hbm.at[pl.ds(rb, QR)], wb.at[s], insem),
        ]

    def drain_in(j, s):
        rb = ebase + j * QR
        pltpu.make_async_copy(src_hbm.at[pl.ds(rb, QR)], sidx.at[s],
                              insem).wait()
        pltpu.make_async_copy(dst_hbm.at[pl.ds(rb, QR)], didx.at[s],
                              insem).wait()
        pltpu.make_async_copy(w_hbm.at[pl.ds(rb, QR)], wb.at[s],
                              insem).wait()

    def issue_gathers(s):
        for q in range(QR):
            pltpu.async_copy(y_hbm.at[sidx.at[s, q]], rows.at[s, q], gsem)
            pltpu.async_copy(dinv_hbm.at[didx.at[s, q]], dgat.at[s, q], gsem)

    def drain_gathers(s):
        for q in range(QR):
            pltpu.make_async_copy(y_hbm.at[sidx.at[s, q]], rows.at[s, q],
                                  gsem).wait()
            pltpu.make_async_copy(dinv_hbm.at[didx.at[s, q]], dgat.at[s, q],
                                  gsem).wait()

    def issue_scatters(s):
        for q in range(QR):
            pltpu.async_copy(rows.at[s, q], acc_sh.at[didx.at[s, q]], ssem,
                             add=True)
            pltpu.async_copy(cprod.at[s, q], c_sh.at[sidx.at[s, q]], ssem,
                             add=True)

    def drain_scatters(s):
        for q in range(QR):
            pltpu.make_async_copy(rows.at[s, q], acc_sh.at[didx.at[s, q]],
                                  ssem).wait()
            pltpu.make_async_copy(cprod.at[s, q], c_sh.at[sidx.at[s, q]],
                                  ssem).wait()

    def compute(s):
        def qbody(q, carry):
            for g in range(8):
                w16 = wb[s, q, pl.ds(g * 16, 16)]
                dv16 = dgat[s, q, pl.ds(g * 16, 16)]
                cprod[s, q, pl.ds(g * 16, 16)] = w16 * dv16
                for e in range(16):
                    k = g * 16 + e
                    rows[s, q, k, :] = rows[s, q, k, :] * w16[e]
            return carry
        lax.fori_loop(0, QR, qbody, 0)

    for d in issue_in(0, 0):
        d.wait()
    issue_gathers(0)

    def group(g, carry):
        j0 = 2 * g
        @pl.when(g > 0)
        def _():
            drain_scatters(1)
        issue_in(j0 + 1, 1)
        drain_gathers(0)
        compute(0)
        issue_scatters(0)
        drain_in(j0 + 1, 1)
        issue_gathers(1)
        drain_scatters(0)
        drain_gathers(1)
        compute(1)
        issue_scatters(1)

        @pl.when(g < NOUT_E // 2 - 1)
        def _():
            issue_in(j0 + 2, 0)
            drain_in(j0 + 2, 0)
            issue_gathers(0)
        return carry

    lax.fori_loop(0, NOUT_E // 2, group, 0)
    drain_scatters(1)
    plsc.subcore_barrier()

    sl = pl.ds(sid * PT, PT)
    pltpu.sync_copy(acc_sh.at[sl], accp.at[cid, sl])

    @pl.when(cid == 0)
    def _():
        pltpu.sync_copy(c_sh.at[sl], c0.at[sl])

    @pl.when(cid == 1)
    def _():
        pltpu.sync_copy(c_sh.at[sl], c1.at[sl])


NR8 = N2 // 8
BR8 = NR8 // GRID


def _dense1_body(xf_ref, w8_ref, d0_ref, d1_ref, e8_ref, dinv_ref, y_ref):
    i = pl.program_id(0)
    deg = d0_ref[...] + d1_ref[...] + 1.0
    dv = lax.rsqrt(deg)
    node = (lax.broadcasted_iota(jnp.int32, (BR8, 8), 0) + i * BR8) * 8 \
        + lax.broadcasted_iota(jnp.int32, (BR8, 8), 1)
    dv = jnp.where(node < N, dv, 0.0)
    dr = jnp.dot(dv, e8_ref[...], preferred_element_type=jnp.float32)
    xw = jnp.dot(xf_ref[...], w8_ref[...], preferred_element_type=jnp.float32)
    dinv_ref[...] = dv
    y_ref[...] = dr * xw


def _dense1(xf, W8, d0v, d1v, E8):
    return pl.pallas_call(
        _dense1_body,
        grid=(GRID,),
        in_specs=[
            pl.BlockSpec((BR8, 128), lambda i: (i, 0)),
            pl.BlockSpec((128, 128), lambda i: (0, 0)),
            pl.BlockSpec((BR8, 8), lambda i: (i, 0)),
            pl.BlockSpec((BR8, 8), lambda i: (i, 0)),
            pl.BlockSpec((8, 128), lambda i: (0, 0)),
        ],
        out_specs=[
            pl.BlockSpec((BR8, 8), lambda i: (i, 0)),
            pl.BlockSpec((BR8, 128), lambda i: (i, 0)),
        ],
        out_shape=[
            jax.ShapeDtypeStruct((NR8, 8), jnp.float32),
            jax.ShapeDtypeStruct((NR8, 128), jnp.float32),
        ],
    )(xf, W8, d0v, d1v, E8)


def _dense2_body(accp_ref, y_ref, dinv_ref, c0_ref, c1_ref, b1r_ref,
                 e8_ref, w2_ref, b2_ref, wlin_ref, blin_ref, out_ref, s_ref):
    i = pl.program_id(0)
    dv = dinv_ref[...]
    dr = jnp.dot(dv, e8_ref[...], preferred_element_type=jnp.float32)
    acc = accp_ref[0] + accp_ref[1]
    h = jax.nn.relu((acc + y_ref[...]) * dr + b1r_ref[...])
    coef = dv * (c0_ref[...] + c1_ref[...] + dv)
    coefr = jnp.dot(coef, e8_ref[...], preferred_element_type=jnp.float32)
    part = jnp.sum(coefr * h, axis=0, keepdims=True)

    @pl.when(i == 0)
    def _():
        s_ref[...] = part

    @pl.when(i > 0)
    def _():
        s_ref[...] = s_ref[...] + part

    @pl.when(i == GRID - 1)
    def _():
        s128 = s_ref[...]
        s16 = s128[:, 0:16]
        for gg in range(1, 8):
            s16 = s16 + s128[:, gg * 16:(gg + 1) * 16]
        g = jnp.dot(s16 / float(N), w2_ref[...],
                    preferred_element_type=jnp.float32) + b2_ref[...]
        out_ref[...] = jnp.dot(g, wlin_ref[...],
                               preferred_element_type=jnp.float32) \
            + blin_ref[...]


def _dense2(accp8, y8, dinv8, c08, c18, b1r, E8, W2, b2, Wlin, blin):
    return pl.pallas_call(
        _dense2_body,
        grid=(GRID,),
        in_specs=[
            pl.BlockSpec((2, BR8, 128), lambda i: (0, i, 0)),
            pl.BlockSpec((BR8, 128), lambda i: (i, 0)),
            pl.BlockSpec((BR8, 8), lambda i: (i, 0)),
            pl.BlockSpec((BR8, 8), lambda i: (i, 0)),
            pl.BlockSpec((BR8, 8), lambda i: (i, 0)),
            pl.BlockSpec((1, 128), lambda i: (0, 0)),
            pl.BlockSpec((8, 128), lambda i: (0, 0)),
            pl.BlockSpec((F, F), lambda i: (0, 0)),
            pl.BlockSpec((1, F), lambda i: (0, 0)),
            pl.BlockSpec((F, 1), lambda i: (0, 0)),
            pl.BlockSpec((1, 1), lambda i: (0, 0)),
        ],
        out_specs=pl.BlockSpec((1, 1), lambda i: (0, 0)),
        out_shape=jax.ShapeDtypeStruct((1, 1), jnp.float32),
        scratch_shapes=[pltpu.VMEM((1, 128), jnp.float32)],
    )(accp8, y8, dinv8, c08, c18, b1r, E8, W2, b2, Wlin, blin)


def kernel(x, edge_index, edge_attr, W1, b1, W2, b2, Wlin, blin):
    src = edge_index[0]
    dst = edge_index[1]
    pad = E2 - E
    src2 = jnp.pad(src, (0, pad)).reshape(-1, 128)
    dst2 = jnp.pad(dst, (0, pad)).reshape(-1, 128)
    w2e = jnp.pad(edge_attr, (0, pad)).reshape(-1, 128)
    xf = jnp.pad(x, ((0, N2 - N), (0, 0))).reshape(NR8, 128)
    eye8 = jnp.eye(8, dtype=jnp.float32)
    W8 = jnp.kron(eye8, W1)
    E8 = jnp.kron(eye8, jnp.ones((1, F), jnp.float32))
    b1r = jnp.tile(b1, 8).reshape(1, 128)

    d0, d1 = _deg_kernel(dst2, w2e)
    dinv8, y8 = _dense1(xf, W8, d0.reshape(NR8, 8), d1.reshape(NR8, 8), E8)
    accp, c0, c1 = _edge_kernel(src2, dst2, w2e,
                                y8.reshape(N2, F), dinv8.reshape(N2))
    out = _dense2(accp.reshape(2, NR8, 128), y8,
                  dinv8, c0.reshape(NR8, 8), c1.reshape(NR8, 8),
                  b1r, E8, W2, b2.reshape(1, F), Wlin, blin.reshape(1, 1))
    return out


# speedup vs baseline: 101.4893x; 1.0053x over previous
import functools

import jax
import jax.numpy as jnp
from jax import lax
from jax.experimental import pallas as pl
from jax.experimental.pallas import tpu as pltpu
from jax.experimental.pallas import tpu_sc as plsc

N = 100000
E = 3200000
F = 16

NC = 2
NS = 16
NW = NC * NS
PT = 6272
N2 = NS * PT
EPT = 100352
ROWS_PT = EPT // 128
NGRP = 98
LAST_NGRP = 87
QR = 4
NOUT_E = EPT // (QR * 128)
BLK = 6272
GRID = N2 // BLK

_mesh = plsc.VectorSubcoreMesh(core_axis_name="c", subcore_axis_name="s")


@functools.partial(
    pl.kernel,
    out_type=(
        jax.ShapeDtypeStruct((N2,), jnp.float32),
        jax.ShapeDtypeStruct((N2,), jnp.float32),
    ),
    mesh=_mesh,
    scratch_types=[
        pltpu.VMEM_SHARED((N2,), jnp.float32),
        pltpu.VMEM((2, QR, 128), jnp.int32),
        pltpu.VMEM((2, QR, 128), jnp.float32),
        pltpu.VMEM((1024,), jnp.float32),
        pltpu.SemaphoreType.DMA,
        pltpu.SemaphoreType.DMA,
    ],
)
def _deg_kernel(dst_hbm, w_hbm, out0, out1, deg_sh, didx, wb, zbuf,
                insem, ssem):
    cid = lax.axis_index("c")
    sid = lax.axis_index("s")
    tid = sid * NC + cid

    z16 = jnp.zeros((16,), jnp.float32)
    for i in range(64):
        zbuf[pl.ds(i * 16, 16)] = z16
    for r in range(6):
        pltpu.sync_copy(zbuf, deg_sh.at[pl.ds(sid * PT + r * 1024, 1024)])
    pltpu.sync_copy(zbuf.at[pl.ds(0, 128)],
                    deg_sh.at[pl.ds(sid * PT + 6144, 128)])
    plsc.subcore_barrier()

    ebase = tid * ROWS_PT

    def issue_in(j, s):
        rb = ebase + j * QR
        return [
            pltpu.async_copy(dst_hbm.at[pl.ds(rb, QR)], didx.at[s], insem),
            pltpu.async_copy(w_hbm.at[pl.ds(rb, QR)], wb.at[s], insem),
        ]

    def drain_in(j, s):
        rb = ebase + j * QR
        pltpu.make_async_copy(dst_hbm.at[pl.ds(rb, QR)], didx.at[s],
                              insem).wait()
        pltpu.make_async_copy(w_hbm.at[pl.ds(rb, QR)], wb.at[s],
                              insem).wait()

    def issue_scatters(s):
        for q in range(QR):
            pltpu.async_copy(wb.at[s, q], deg_sh.at[didx.at[s, q]], ssem,
                             add=True)

    def drain_scatters(s):
        for q in range(QR):
            pltpu.make_async_copy(wb.at[s, q], deg_sh.at[didx.at[s, q]],
                                  ssem).wait()

    for d in issue_in(0, 0):
        d.wait()

    ngrp = lax.select(tid == NW - 1, LAST_NGRP, NGRP)

    def group(g, carry):
        j0 = 2 * g

        @pl.when(g > 0)
        def _():
            drain_scatters(1)
        issue_in(j0 + 1, 1)
        issue_scatters(0)
        drain_in(j0 + 1, 1)
        issue_scatters(1)
        drain_scatters(0)

        @pl.when(g < ngrp - 1)
        def _():
            issue_in(j0 + 2, 0)
            drain_in(j0 + 2, 0)
        return carry

    lax.fori_loop(0, ngrp, group, 0)
    drain_scatters(1)
    plsc.subcore_barrier()

    sl = pl.ds(sid * PT, PT)

    @pl.when(cid == 0)
    def _():
        pltpu.sync_copy(deg_sh.at[sl], out0.at[sl])

    @pl.when(cid == 1)
    def _():
        pltpu.sync_copy(deg_sh.at[sl], out1.at[sl])


@functools.partial(
    pl.kernel,
    out_type=(
        jax.ShapeDtypeStruct((2, N2, F), jnp.float32),
        jax.ShapeDtypeStruct((N2,), jnp.float32),
        jax.ShapeDtypeStruct((N2,), jnp.float32),
    ),
    mesh=_mesh,
    scratch_types=[
        pltpu.VMEM_SHARED((N2, F), jnp.float32),
        pltpu.VMEM_SHARED((N2,), jnp.float32),
        pltpu.VMEM((2, QR, 128), jnp.float32),
        pltpu.VMEM((2, QR, 128), jnp.int32),
        pltpu.VMEM((2, QR, 128), jnp.int32),
        pltpu.VMEM((2, QR, 128), jnp.float32),
        pltpu.VMEM((2, QR, 128, F), jnp.float32),
        pltpu.VMEM((2, QR, 128), jnp.float32),
        pltpu.VMEM((512,), jnp.float32),
        pltpu.VMEM((64, F), jnp.float32),
        pltpu.SemaphoreType.DMA,
        pltpu.SemaphoreType.DMA,
        pltpu.SemaphoreType.DMA,
    ],
    compiler_params=pltpu.CompilerParams(use_tc_tiling_on_sc=False),
)
def _edge_kernel(src_hbm, dst_hbm, w_hbm, y_hbm, dinv_hbm,
                 accp, c0, c1,
                 acc_sh, c_sh, dgat, sidx, didx, wb, rows, cprod,
                 zbuf, zrows, insem, gsem, ssem):
    cid = lax.axis_index("c")
    sid = lax.axis_index("s")
    tid = sid * NC + cid

    z16 = jnp.zeros((16,), jnp.float32)
    for i in range(64):
        zrows[i, :] = z16
    for i in range(32):
        zbuf[pl.ds(i * 16, 16)] = z16
    for r in range(98):
        pltpu.sync_copy(zrows, acc_sh.at[pl.ds(sid * PT + r * 64, 64)])
    for r in range(12):
        pltpu.sync_copy(zbuf, c_sh.at[pl.ds(sid * PT + r * 512, 512)])
    pltpu.sync_copy(zbuf.at[pl.ds(0, 128)],
                    c_sh.at[pl.ds(sid * PT + 6144, 128)])
    plsc.subcore_barrier()

    ebase = tid * ROWS_PT

    def issue_in(j, s):
        rb = ebase + j * QR
        return [
            pltpu.async_copy(src_hbm.at[pl.ds(rb, QR)], sidx.at[s], insem),
            pltpu.async_copy(dst_hbm.at[pl.ds(rb, QR)], didx.at[s], insem),
            pltpu.async_copy(w_hbm.at[pl.ds(rb, QR)], wb.at[s], insem),
        ]

    def drain_in(j, s):
        rb = ebase + j * QR
        pltpu.make_async_copy(src_hbm.at[pl.ds(rb, QR)], sidx.at[s],
                              insem).wait()
        pltpu.make_async_copy(dst_hbm.at[pl.ds(rb, QR)], didx.at[s],
                              insem).wait()
        pltpu.make_async_copy(w_hbm.at[pl.ds(rb, QR)], wb.at[s],
                              insem).wait()

    def issue_gathers(s):
        for q in range(QR):
            pltpu.async_copy(y_hbm.at[sidx.at[s, q]], rows.at[s, q], gsem)
            pltpu.async_copy(dinv_hbm.at[didx.at[s, q]], dgat.at[s, q], gsem)

    def drain_gathers(s):
        for q in range(QR):
            pltpu.make_async_copy(y_hbm.at[sidx.at[s, q]], rows.at[s, q],
                                  gsem).wait()
            pltpu.make_async_copy(dinv_hbm.at[didx.at[s, q]], dgat.at[s, q],
                                  gsem).wait()

    def issue_scatters(s):
        for q in range(QR):
            pltpu.async_copy(rows.at[s, q], acc_sh.at[didx.at[s, q]], ssem,
                             add=True)
            pltpu.async_copy(cprod.at[s, q], c_sh.at[sidx.at[s, q]], ssem,
                             add=True)

    def drain_scatters(s):
        for q in range(QR):
            pltpu.make_async_copy(rows.at[s, q], acc_sh.at[didx.at[s, q]],
                                  ssem).wait()
            pltpu.make_async_copy(cprod.at[s, q], c_sh.at[sidx.at[s, q]],
                                  ssem).wait()

    def compute(s):
        def qbody(q, carry):
            for g in range(8):
                w16 = wb[s, q, pl.ds(g * 16, 16)]
                dv16 = dgat[s, q, pl.ds(g * 16, 16)]
                cprod[s, q, pl.ds(g * 16, 16)] = w16 * dv16
                for e in range(16):
                    k = g * 16 + e
                    rows[s, q, k, :] = rows[s, q, k, :] * w16[e]
            return carry
        lax.fori_loop(0, QR, qbody, 0)

    for d in issue_in(0, 0):
        d.wait()
    issue_gathers(0)

    ngrp = lax.select(tid == NW - 1, LAST_NGRP, NGRP)

    def group(g, carry):
        j0 = 2 * g
        @pl.when(g > 0)
        def _():
            drain_scatters(1)
        issue_in(j0 + 1, 1)
        drain_gathers(0)
        compute(0)
        issue_scatters(0)
        drain_in(j0 + 1, 1)
        issue_gathers(1)
        drain_scatters(0)
        drain_gathers(1)
        compute(1)
        issue_scatters(1)

        @pl.when(g < ngrp - 1)
        def _():
            issue_in(j0 + 2, 0)
            drain_in(j0 + 2, 0)
            issue_gathers(0)
        return carry

    lax.fori_loop(0, ngrp, group, 0)
    drain_scatters(1)
    plsc.subcore_barrier()

    sl = pl.ds(sid * PT, PT)
    pltpu.sync_copy(acc_sh.at[sl], accp.at[cid, sl])

    @pl.when(cid == 0)
    def _():
        pltpu.sync_copy(c_sh.at[sl], c0.at[sl])

    @pl.when(cid == 1)
    def _():
        pltpu.sync_copy(c_sh.at[sl], c1.at[sl])


NR8 = N2 // 8
BR8 = NR8 // GRID


def _dense1_body(xf_ref, w8_ref, d0_ref, d1_ref, e8_ref, dinv_ref, y_ref):
    i = pl.program_id(0)
    deg = d0_ref[...] + d1_ref[...] + 1.0
    dv = lax.rsqrt(deg)
    node = (lax.broadcasted_iota(jnp.int32, (BR8, 8), 0) + i * BR8) * 8 \
        + lax.broadcasted_iota(jnp.int32, (BR8, 8), 1)
    dv = jnp.where(node < N, dv, 0.0)
    dr = jnp.dot(dv, e8_ref[...], preferred_element_type=jnp.float32)
    xw = jnp.dot(xf_ref[...], w8_ref[...], preferred_element_type=jnp.float32)
    dinv_ref[...] = dv
    y_ref[...] = dr * xw


def _dense1(xf, W8, d0v, d1v, E8):
    return pl.pallas_call(
        _dense1_body,
        grid=(GRID,),
        in_specs=[
            pl.BlockSpec((BR8, 128), lambda i: (i, 0)),
            pl.BlockSpec((128, 128), lambda i: (0, 0)),
            pl.BlockSpec((BR8, 8), lambda i: (i, 0)),
            pl.BlockSpec((BR8, 8), lambda i: (i, 0)),
            pl.BlockSpec((8, 128), lambda i: (0, 0)),
        ],
        out_specs=[
            pl.BlockSpec((BR8, 8), lambda i: (i, 0)),
            pl.BlockSpec((BR8, 128), lambda i: (i, 0)),
        ],
        out_shape=[
            jax.ShapeDtypeStruct((NR8, 8), jnp.float32),
            jax.ShapeDtypeStruct((NR8, 128), jnp.float32),
        ],
    )(xf, W8, d0v, d1v, E8)


def _dense2_body(accp_ref, y_ref, dinv_ref, c0_ref, c1_ref, b1r_ref,
                 e8_ref, w2_ref, b2_ref, wlin_ref, blin_ref, out_ref, s_ref):
    i = pl.program_id(0)
    dv = dinv_ref[...]
    dr = jnp.dot(dv, e8_ref[...], preferred_element_type=jnp.float32)
    acc = accp_ref[0] + accp_ref[1]
    h = jax.nn.relu((acc + y_ref[...]) * dr + b1r_ref[...])
    coef = dv * (c0_ref[...] + c1_ref[...] + dv)
    coefr = jnp.dot(coef, e8_ref[...], preferred_element_type=jnp.float32)
    part = jnp.sum(coefr * h, axis=0, keepdims=True)

    @pl.when(i == 0)
    def _():
        s_ref[...] = part

    @pl.when(i > 0)
    def _():
        s_ref[...] = s_ref[...] + part

    @pl.when(i == GRID - 1)
    def _():
        s128 = s_ref[...]
        s16 = s128[:, 0:16]
        for gg in range(1, 8):
            s16 = s16 + s128[:, gg * 16:(gg + 1) * 16]
        g = jnp.dot(s16 / float(N), w2_ref[...],
                    preferred_element_type=jnp.float32) + b2_ref[...]
        out_ref[...] = jnp.dot(g, wlin_ref[...],
                               preferred_element_type=jnp.float32) \
            + blin_ref[...]


def _dense2(accp8, y8, dinv8, c08, c18, b1r, E8, W2, b2, Wlin, blin):
    return pl.pallas_call(
        _dense2_body,
        grid=(GRID,),
        in_specs=[
            pl.BlockSpec((2, BR8, 128), lambda i: (0, i, 0)),
            pl.BlockSpec((BR8, 128), lambda i: (i, 0)),
            pl.BlockSpec((BR8, 8), lambda i: (i, 0)),
            pl.BlockSpec((BR8, 8), lambda i: (i, 0)),
            pl.BlockSpec((BR8, 8), lambda i: (i, 0)),
            pl.BlockSpec((1, 128), lambda i: (0, 0)),
            pl.BlockSpec((8, 128), lambda i: (0, 0)),
            pl.BlockSpec((F, F), lambda i: (0, 0)),
            pl.BlockSpec((1, F), lambda i: (0, 0)),
            pl.BlockSpec((F, 1), lambda i: (0, 0)),
            pl.BlockSpec((1, 1), lambda i: (0, 0)),
        ],
        out_specs=pl.BlockSpec((1, 1), lambda i: (0, 0)),
        out_shape=jax.ShapeDtypeStruct((1, 1), jnp.float32),
        scratch_shapes=[pltpu.VMEM((1, 128), jnp.float32)],
    )(accp8, y8, dinv8, c08, c18, b1r, E8, W2, b2, Wlin, blin)


def kernel(x, edge_index, edge_attr, W1, b1, W2, b2, Wlin, blin):
    src2 = edge_index[0].reshape(-1, 128)
    dst2 = edge_index[1].reshape(-1, 128)
    w2e = edge_attr.reshape(-1, 128)
    xf = jnp.pad(x, ((0, N2 - N), (0, 0))).reshape(NR8, 128)
    eye8 = jnp.eye(8, dtype=jnp.float32)
    W8 = jnp.kron(eye8, W1)
    E8 = jnp.kron(eye8, jnp.ones((1, F), jnp.float32))
    b1r = jnp.tile(b1, 8).reshape(1, 128)

    d0, d1 = _deg_kernel(dst2, w2e)
    dinv8, y8 = _dense1(xf, W8, d0.reshape(NR8, 8), d1.reshape(NR8, 8), E8)
    accp, c0, c1 = _edge_kernel(src2, dst2, w2e,
                                y8.reshape(N2, F), dinv8.reshape(N2))
    out = _dense2(accp.reshape(2, NR8, 128), y8,
                  dinv8, c0.reshape(NR8, 8), c1.reshape(NR8, 8),
                  b1r, E8, W2, b2.reshape(1, F), Wlin, blin.reshape(1, 1))
    return out


# speedup vs baseline: 101.4985x; 1.0001x over previous
import functools

import jax
import jax.numpy as jnp
from jax import lax
from jax.experimental import pallas as pl
from jax.experimental.pallas import tpu as pltpu
from jax.experimental.pallas import tpu_sc as plsc

N = 100000
E = 3200000
F = 16

NC = 2
NS = 16
NW = NC * NS
PT = 6272
N2 = NS * PT
EPT = 100352
ROWS_PT = EPT // 128
NGRP = 98
LAST_NGRP = 87
QR = 4
NOUT_E = EPT // (QR * 128)
BLK = 6272
GRID = N2 // BLK

_mesh = plsc.VectorSubcoreMesh(core_axis_name="c", subcore_axis_name="s")


@functools.partial(
    pl.kernel,
    out_type=(
        jax.ShapeDtypeStruct((N2,), jnp.float32),
        jax.ShapeDtypeStruct((N2,), jnp.float32),
    ),
    mesh=_mesh,
    scratch_types=[
        pltpu.VMEM_SHARED((N2,), jnp.float32),
        pltpu.VMEM((2, QR, 128), jnp.int32),
        pltpu.VMEM((2, QR, 128), jnp.float32),
        pltpu.VMEM((1024,), jnp.float32),
        pltpu.SemaphoreType.DMA,
        pltpu.SemaphoreType.DMA,
    ],
)
def _deg_kernel(dst_hbm, w_hbm, out0, out1, deg_sh, didx, wb, zbuf,
                insem, ssem):
    cid = lax.axis_index("c")
    sid = lax.axis_index("s")
    tid = sid * NC + cid

    z16 = jnp.zeros((16,), jnp.float32)
    for i in range(64):
        zbuf[pl.ds(i * 16, 16)] = z16
    for r in range(6):
        pltpu.sync_copy(zbuf, deg_sh.at[pl.ds(sid * PT + r * 1024, 1024)])
    pltpu.sync_copy(zbuf.at[pl.ds(0, 128)],
                    deg_sh.at[pl.ds(sid * PT + 6144, 128)])
    plsc.subcore_barrier()

    ebase = tid * ROWS_PT

    def issue_in(j, s):
        rb = ebase + j * QR
        return [
            pltpu.async_copy(dst_hbm.at[pl.ds(rb, QR)], didx.at[s], insem),
            pltpu.async_copy(w_hbm.at[pl.ds(rb, QR)], wb.at[s], insem),
        ]

    def drain_in(j, s):
        rb = ebase + j * QR
        pltpu.make_async_copy(dst_hbm.at[pl.ds(rb, QR)], didx.at[s],
                              insem).wait()
        pltpu.make_async_copy(w_hbm.at[pl.ds(rb, QR)], wb.at[s],
                              insem).wait()

    def issue_scatters(s):
        for q in range(QR):
            pltpu.async_copy(wb.at[s, q], deg_sh.at[didx.at[s, q]], ssem,
                             add=True)

    def drain_scatters(s):
        for q in range(QR):
            pltpu.make_async_copy(wb.at[s, q], deg_sh.at[didx.at[s, q]],
                                  ssem).wait()

    for d in issue_in(0, 0):
        d.wait()

    ngrp = lax.select(tid == NW - 1, LAST_NGRP, NGRP)

    def group(g, carry):
        j0 = 2 * g

        @pl.when(g > 0)
        def _():
            drain_scatters(1)
        issue_in(j0 + 1, 1)
        issue_scatters(0)
        drain_in(j0 + 1, 1)
        issue_scatters(1)
        drain_scatters(0)

        @pl.when(g < ngrp - 1)
        def _():
            issue_in(j0 + 2, 0)
            drain_in(j0 + 2, 0)
        return carry

    lax.fori_loop(0, ngrp, group, 0)
    drain_scatters(1)
    plsc.subcore_barrier()

    sl = pl.ds(sid * PT, PT)

    @pl.when(cid == 0)
    def _():
        pltpu.sync_copy(deg_sh.at[sl], out0.at[sl])

    @pl.when(cid == 1)
    def _():
        pltpu.sync_copy(deg_sh.at[sl], out1.at[sl])


@functools.partial(
    pl.kernel,
    out_type=(
        jax.ShapeDtypeStruct((2, N2, F), jnp.float32),
        jax.ShapeDtypeStruct((N2,), jnp.float32),
        jax.ShapeDtypeStruct((N2,), jnp.float32),
    ),
    mesh=_mesh,
    scratch_types=[
        pltpu.VMEM_SHARED((N2, F), jnp.float32),
        pltpu.VMEM_SHARED((N2,), jnp.float32),
        pltpu.VMEM((2, QR, 128), jnp.float32),
        pltpu.VMEM((2, QR, 128), jnp.int32),
        pltpu.VMEM((2, QR, 128), jnp.int32),
        pltpu.VMEM((2, QR, 128), jnp.float32),
        pltpu.VMEM((2, QR, 128, F), jnp.float32),
        pltpu.VMEM((2, QR, 128), jnp.float32),
        pltpu.VMEM((512,), jnp.float32),
        pltpu.VMEM((64, F), jnp.float32),
        pltpu.SemaphoreType.DMA,
        pltpu.SemaphoreType.DMA,
        pltpu.SemaphoreType.DMA,
    ],
    compiler_params=pltpu.CompilerParams(use_tc_tiling_on_sc=False),
)
def _edge_kernel(src_hbm, dst_hbm, w_hbm, y_hbm, dinv_hbm,
                 accp, c0, c1,
                 acc_sh, c_sh, dgat, sidx, didx, wb, rows, cprod,
                 zbuf, zrows, insem, gsem, ssem):
    cid = lax.axis_index("c")
    sid = lax.axis_index("s")
    tid = sid * NC + cid

    z16 = jnp.zeros((16,), jnp.float32)
    for i in range(64):
        zrows[i, :] = z16
    for i in range(32):
        zbuf[pl.ds(i * 16, 16)] = z16
    for r in range(98):
        pltpu.sync_copy(zrows, acc_sh.at[pl.ds(sid * PT + r * 64, 64)])
    for r in range(12):
        pltpu.sync_copy(zbuf, c_sh.at[pl.ds(sid * PT + r * 512, 512)])
    pltpu.sync_copy(zbuf.at[pl.ds(0, 128)],
                    c_sh.at[pl.ds(sid * PT + 6144, 128)])
    plsc.subcore_barrier()

    ebase = tid * ROWS_PT

    def issue_in(j, s):
        rb = ebase + j * QR
        return [
            pltpu.async_copy(src_hbm.at[pl.ds(rb, QR)], sidx.at[s], insem),
            pltpu.async_copy(dst_hbm.at[pl.ds(rb, QR)], didx.at[s], insem),
            pltpu.async_copy(w_hbm.at[pl.ds(rb, QR)], wb.at[s], insem),
        ]

    def drain_in(j, s):
        rb = ebase + j * QR
        pltpu.make_async_copy(src_hbm.at[pl.ds(rb, QR)], sidx.at[s],
                              insem).wait()
        pltpu.make_async_copy(dst_hbm.at[pl.ds(rb, QR)], didx.at[s],
                              insem).wait()
        pltpu.make_async_copy(w_hbm.at[pl.ds(rb, QR)], wb.at[s],
                              insem).wait()

    def issue_gathers(s):
        for q in range(QR):
            pltpu.async_copy(y_hbm.at[sidx.at[s, q]], rows.at[s, q], gsem)
            pltpu.async_copy(dinv_hbm.at[didx.at[s, q]], dgat.at[s, q], gsem)

    def drain_gathers(s):
        for q in range(QR):
            pltpu.make_async_copy(y_hbm.at[sidx.at[s, q]], rows.at[s, q],
                                  gsem).wait()
            pltpu.make_async_copy(dinv_hbm.at[didx.at[s, q]], dgat.at[s, q],
                                  gsem).wait()

    def issue_scatters(s):
        for q in range(QR):
            pltpu.async_copy(rows.at[s, q], acc_sh.at[didx.at[s, q]], ssem,
                             add=True)
            pltpu.async_copy(cprod.at[s, q], c_sh.at[sidx.at[s, q]], ssem,
                             add=True)

    def drain_scatters(s):
        for q in range(QR):
            pltpu.make_async_copy(rows.at[s, q], acc_sh.at[didx.at[s, q]],
                                  ssem).wait()
            pltpu.make_async_copy(cprod.at[s, q], c_sh.at[sidx.at[s, q]],
                                  ssem).wait()

    splat_idx = [jnp.full((16,), e, jnp.int32) for e in range(16)]

    def compute(s):
        def qbody(q, carry):
            for g in range(8):
                w16 = wb[s, q, pl.ds(g * 16, 16)]
                dv16 = dgat[s, q, pl.ds(g * 16, 16)]
                cprod[s, q, pl.ds(g * 16, 16)] = w16 * dv16
                for e in range(16):
                    k = g * 16 + e
                    ws = jnp.take(w16, splat_idx[e])
                    rows[s, q, k, :] = rows[s, q, k, :] * ws
            return carry
        lax.fori_loop(0, QR, qbody, 0)

    for d in issue_in(0, 0):
        d.wait()
    issue_gathers(0)

    ngrp = lax.select(tid == NW - 1, LAST_NGRP, NGRP)

    def group(g, carry):
        j0 = 2 * g
        @pl.when(g > 0)
        def _():
            drain_scatters(1)
        issue_in(j0 + 1, 1)
        drain_gathers(0)
        compute(0)
        issue_scatters(0)
        drain_in(j0 + 1, 1)
        issue_gathers(1)
        drain_scatters(0)
        drain_gathers(1)
        compute(1)
        issue_scatters(1)

        @pl.when(g < ngrp - 1)
        def _():
            issue_in(j0 + 2, 0)
            drain_in(j0 + 2, 0)
            issue_gathers(0)
        return carry

    lax.fori_loop(0, ngrp, group, 0)
    drain_scatters(1)
    plsc.subcore_barrier()

    sl = pl.ds(sid * PT, PT)
    pltpu.sync_copy(acc_sh.at[sl], accp.at[cid, sl])

    @pl.when(cid == 0)
    def _():
        pltpu.sync_copy(c_sh.at[sl], c0.at[sl])

    @pl.when(cid == 1)
    def _():
        pltpu.sync_copy(c_sh.at[sl], c1.at[sl])


NR8 = N2 // 8
BR8 = NR8 // GRID


def _dense1_body(xf_ref, w8_ref, d0_ref, d1_ref, e8_ref, dinv_ref, y_ref):
    i = pl.program_id(0)
    deg = d0_ref[...] + d1_ref[...] + 1.0
    dv = lax.rsqrt(deg)
    node = (lax.broadcasted_iota(jnp.int32, (BR8, 8), 0) + i * BR8) * 8 \
        + lax.broadcasted_iota(jnp.int32, (BR8, 8), 1)
    dv = jnp.where(node < N, dv, 0.0)
    dr = jnp.dot(dv, e8_ref[...], preferred_element_type=jnp.float32)
    xw = jnp.dot(xf_ref[...], w8_ref[...], preferred_element_type=jnp.float32)
    dinv_ref[...] = dv
    y_ref[...] = dr * xw


def _dense1(xf, W8, d0v, d1v, E8):
    return pl.pallas_call(
        _dense1_body,
        grid=(GRID,),
        in_specs=[
            pl.BlockSpec((BR8, 128), lambda i: (i, 0)),
            pl.BlockSpec((128, 128), lambda i: (0, 0)),
            pl.BlockSpec((BR8, 8), lambda i: (i, 0)),
            pl.BlockSpec((BR8, 8), lambda i: (i, 0)),
            pl.BlockSpec((8, 128), lambda i: (0, 0)),
        ],
        out_specs=[
            pl.BlockSpec((BR8, 8), lambda i: (i, 0)),
            pl.BlockSpec((BR8, 128), lambda i: (i, 0)),
        ],
        out_shape=[
            jax.ShapeDtypeStruct((NR8, 8), jnp.float32),
            jax.ShapeDtypeStruct((NR8, 128), jnp.float32),
        ],
    )(xf, W8, d0v, d1v, E8)


def _dense2_body(accp_ref, y_ref, dinv_ref, c0_ref, c1_ref, b1r_ref,
                 e8_ref, w2_ref, b2_ref, wlin_ref, blin_ref, out_ref, s_ref):
    i = pl.program_id(0)
    dv = dinv_ref[...]
    dr = jnp.dot(dv, e8_ref[...], preferred_element_type=jnp.float32)
    acc = accp_ref[0] + accp_ref[1]
    h = jax.nn.relu((acc + y_ref[...]) * dr + b1r_ref[...])
    coef = dv * (c0_ref[...] + c1_ref[...] + dv)
    coefr = jnp.dot(coef, e8_ref[...], preferred_element_type=jnp.float32)
    part = jnp.sum(coefr * h, axis=0, keepdims=True)

    @pl.when(i == 0)
    def _():
        s_ref[...] = part

    @pl.when(i > 0)
    def _():
        s_ref[...] = s_ref[...] + part

    @pl.when(i == GRID - 1)
    def _():
        s128 = s_ref[...]
        s16 = s128[:, 0:16]
        for gg in range(1, 8):
            s16 = s16 + s128[:, gg * 16:(gg + 1) * 16]
        g = jnp.dot(s16 / float(N), w2_ref[...],
                    preferred_element_type=jnp.float32) + b2_ref[...]
        out_ref[...] = jnp.dot(g, wlin_ref[...],
                               preferred_element_type=jnp.float32) \
            + blin_ref[...]


def _dense2(accp8, y8, dinv8, c08, c18, b1r, E8, W2, b2, Wlin, blin):
    return pl.pallas_call(
        _dense2_body,
        grid=(GRID,),
        in_specs=[
            pl.BlockSpec((2, BR8, 128), lambda i: (0, i, 0)),
            pl.BlockSpec((BR8, 128), lambda i: (i, 0)),
            pl.BlockSpec((BR8, 8), lambda i: (i, 0)),
            pl.BlockSpec((BR8, 8), lambda i: (i, 0)),
            pl.BlockSpec((BR8, 8), lambda i: (i, 0)),
            pl.BlockSpec((1, 128), lambda i: (0, 0)),
            pl.BlockSpec((8, 128), lambda i: (0, 0)),
            pl.BlockSpec((F, F), lambda i: (0, 0)),
            pl.BlockSpec((1, F), lambda i: (0, 0)),
            pl.BlockSpec((F, 1), lambda i: (0, 0)),
            pl.BlockSpec((1, 1), lambda i: (0, 0)),
        ],
        out_specs=pl.BlockSpec((1, 1), lambda i: (0, 0)),
        out_shape=jax.ShapeDtypeStruct((1, 1), jnp.float32),
        scratch_shapes=[pltpu.VMEM((1, 128), jnp.float32)],
    )(accp8, y8, dinv8, c08, c18, b1r, E8, W2, b2, Wlin, blin)


def kernel(x, edge_index, edge_attr, W1, b1, W2, b2, Wlin, blin):
    src2 = edge_index[0].reshape(-1, 128)
    dst2 = edge_index[1].reshape(-1, 128)
    w2e = edge_attr.reshape(-1, 128)
    xf = jnp.pad(x, ((0, N2 - N), (0, 0))).reshape(NR8, 128)
    eye8 = jnp.eye(8, dtype=jnp.float32)
    W8 = jnp.kron(eye8, W1)
    E8 = jnp.kron(eye8, jnp.ones((1, F), jnp.float32))
    b1r = jnp.tile(b1, 8).reshape(1, 128)

    d0, d1 = _deg_kernel(dst2, w2e)
    dinv8, y8 = _dense1(xf, W8, d0.reshape(NR8, 8), d1.reshape(NR8, 8), E8)
    accp, c0, c1 = _edge_kernel(src2, dst2, w2e,
                                y8.reshape(N2, F), dinv8.reshape(N2))
    out = _dense2(accp.reshape(2, NR8, 128), y8,
                  dinv8, c0.reshape(NR8, 8), c1.reshape(NR8, 8),
                  b1r, E8, W2, b2.reshape(1, F), Wlin, blin.reshape(1, 1))
    return out


# speedup vs baseline: 101.5632x; 1.0006x over previous
import functools

import jax
import jax.numpy as jnp
from jax import lax
from jax.experimental import pallas as pl
from jax.experimental.pallas import tpu as pltpu
from jax.experimental.pallas import tpu_sc as plsc

N = 100000
E = 3200000
F = 16

NC = 2
NS = 16
NW = NC * NS
PT = 6272
N2 = NS * PT
EPT = 100352
ROWS_PT = EPT // 128
NGRP = 98
LAST_NGRP = 87
QR = 4
NOUT_E = EPT // (QR * 128)
BLK = 6272
GRID = N2 // BLK

_mesh = plsc.VectorSubcoreMesh(core_axis_name="c", subcore_axis_name="s")


@functools.partial(
    pl.kernel,
    out_type=(
        jax.ShapeDtypeStruct((N2,), jnp.float32),
        jax.ShapeDtypeStruct((N2,), jnp.float32),
    ),
    mesh=_mesh,
    scratch_types=[
        pltpu.VMEM_SHARED((N2,), jnp.float32),
        pltpu.VMEM((2, QR, 128), jnp.int32),
        pltpu.VMEM((2, QR, 128), jnp.float32),
        pltpu.VMEM((1024,), jnp.float32),
        pltpu.SemaphoreType.DMA,
        pltpu.SemaphoreType.DMA,
    ],
)
def _deg_kernel(dst_hbm, w_hbm, out0, out1, deg_sh, didx, wb, zbuf,
                insem, ssem):
    cid = lax.axis_index("c")
    sid = lax.axis_index("s")
    tid = sid * NC + cid

    z16 = jnp.zeros((16,), jnp.float32)
    for i in range(64):
        zbuf[pl.ds(i * 16, 16)] = z16
    for r in range(6):
        pltpu.sync_copy(zbuf, deg_sh.at[pl.ds(sid * PT + r * 1024, 1024)])
    pltpu.sync_copy(zbuf.at[pl.ds(0, 128)],
                    deg_sh.at[pl.ds(sid * PT + 6144, 128)])
    plsc.subcore_barrier()

    ebase = tid * ROWS_PT

    def issue_in(j, s):
        rb = ebase + j * QR
        return [
            pltpu.async_copy(dst_hbm.at[pl.ds(rb, QR)], didx.at[s], insem),
            pltpu.async_copy(w_hbm.at[pl.ds(rb, QR)], wb.at[s], insem),
        ]

    def drain_in(j, s):
        rb = ebase + j * QR
        pltpu.make_async_copy(dst_hbm.at[pl.ds(rb, QR)], didx.at[s],
                              insem).wait()
        pltpu.make_async_copy(w_hbm.at[pl.ds(rb, QR)], wb.at[s],
                              insem).wait()

    def issue_scatters(s):
        for q in range(QR):
            pltpu.async_copy(wb.at[s, q], deg_sh.at[didx.at[s, q]], ssem,
                             add=True)

    def drain_scatters(s):
        for q in range(QR):
            pltpu.make_async_copy(wb.at[s, q], deg_sh.at[didx.at[s, q]],
                                  ssem).wait()

    for d in issue_in(0, 0):
        d.wait()

    ngrp = lax.select(tid == NW - 1, LAST_NGRP, NGRP)

    def group(g, carry):
        j0 = 2 * g

        @pl.when(g > 0)
        def _():
            drain_scatters(1)
        issue_in(j0 + 1, 1)
        issue_scatters(0)
        drain_in(j0 + 1, 1)
        issue_scatters(1)
        drain_scatters(0)

        @pl.when(g < ngrp - 1)
        def _():
            issue_in(j0 + 2, 0)
            drain_in(j0 + 2, 0)
        return carry

    lax.fori_loop(0, ngrp, group, 0)
    drain_scatters(1)
    plsc.subcore_barrier()

    sl = pl.ds(sid * PT, PT)

    @pl.when(cid == 0)
    def _():
        pltpu.sync_copy(deg_sh.at[sl], out0.at[sl])

    @pl.when(cid == 1)
    def _():
        pltpu.sync_copy(deg_sh.at[sl], out1.at[sl])


@functools.partial(
    pl.kernel,
    out_type=(
        jax.ShapeDtypeStruct((2, N2, F), jnp.float32),
        jax.ShapeDtypeStruct((N2,), jnp.float32),
        jax.ShapeDtypeStruct((N2,), jnp.float32),
    ),
    mesh=_mesh,
    scratch_types=[
        pltpu.VMEM_SHARED((N2, F), jnp.float32),
        pltpu.VMEM_SHARED((N2,), jnp.float32),
        pltpu.VMEM((2, QR, 128), jnp.float32),
        pltpu.VMEM((2, QR, 128), jnp.int32),
        pltpu.VMEM((2, QR, 128), jnp.int32),
        pltpu.VMEM((2, QR, 128), jnp.float32),
        pltpu.VMEM((2, QR, 128, F), jnp.float32),
        pltpu.VMEM((2, QR, 128), jnp.float32),
        pltpu.VMEM((512,), jnp.float32),
        pltpu.VMEM((64, F), jnp.float32),
        pltpu.SemaphoreType.DMA,
        pltpu.SemaphoreType.DMA,
        pltpu.SemaphoreType.DMA,
    ],
    compiler_params=pltpu.CompilerParams(use_tc_tiling_on_sc=False),
)
def _edge_kernel(src_hbm, dst_hbm, w_hbm, y_hbm, dinv_hbm,
                 accp, c0, c1,
                 acc_sh, c_sh, dgat, sidx, didx, wb, rows, cprod,
                 zbuf, zrows, insem, gsem, ssem):
    cid = lax.axis_index("c")
    sid = lax.axis_index("s")
    tid = sid * NC + cid

    z16 = jnp.zeros((16,), jnp.float32)
    for i in range(64):
        zrows[i, :] = z16
    for i in range(32):
        zbuf[pl.ds(i * 16, 16)] = z16
    for r in range(98):
        pltpu.sync_copy(zrows, acc_sh.at[pl.ds(sid * PT + r * 64, 64)])
    for r in range(12):
        pltpu.sync_copy(zbuf, c_sh.at[pl.ds(sid * PT + r * 512, 512)])
    pltpu.sync_copy(zbuf.at[pl.ds(0, 128)],
                    c_sh.at[pl.ds(sid * PT + 6144, 128)])
    plsc.subcore_barrier()

    ebase = tid * ROWS_PT

    def issue_in(j, s):
        rb = ebase + j * QR
        return [
            pltpu.async_copy(src_hbm.at[pl.ds(rb, QR)], sidx.at[s], insem),
            pltpu.async_copy(dst_hbm.at[pl.ds(rb, QR)], didx.at[s], insem),
            pltpu.async_copy(w_hbm.at[pl.ds(rb, QR)], wb.at[s], insem),
        ]

    def drain_in(j, s):
        rb = ebase + j * QR
        pltpu.make_async_copy(src_hbm.at[pl.ds(rb, QR)], sidx.at[s],
                              insem).wait()
        pltpu.make_async_copy(dst_hbm.at[pl.ds(rb, QR)], didx.at[s],
                              insem).wait()
        pltpu.make_async_copy(w_hbm.at[pl.ds(rb, QR)], wb.at[s],
                              insem).wait()

    def issue_gathers(s):
        for q in range(QR):
            pltpu.async_copy(y_hbm.at[sidx.at[s, q]], rows.at[s, q], gsem)
            pltpu.async_copy(dinv_hbm.at[didx.at[s, q]], dgat.at[s, q], gsem)

    def drain_gathers(s):
        for q in range(QR):
            pltpu.make_async_copy(y_hbm.at[sidx.at[s, q]], rows.at[s, q],
                                  gsem).wait()
            pltpu.make_async_copy(dinv_hbm.at[didx.at[s, q]], dgat.at[s, q],
                                  gsem).wait()

    def issue_scatters(s):
        for q in range(QR):
            pltpu.async_copy(rows.at[s, q], acc_sh.at[didx.at[s, q]], ssem,
                             add=True)
            pltpu.async_copy(cprod.at[s, q], c_sh.at[sidx.at[s, q]], ssem,
                             add=True)

    def drain_scatters(s):
        for q in range(QR):
            pltpu.make_async_copy(rows.at[s, q], acc_sh.at[didx.at[s, q]],
                                  ssem).wait()
            pltpu.make_async_copy(cprod.at[s, q], c_sh.at[sidx.at[s, q]],
                                  ssem).wait()

    def compute(s):
        def qbody(q, carry):
            for g in range(8):
                w16 = wb[s, q, pl.ds(g * 16, 16)]
                dv16 = dgat[s, q, pl.ds(g * 16, 16)]
                cprod[s, q, pl.ds(g * 16, 16)] = w16 * dv16
                for e in range(16):
                    k = g * 16 + e
                    rows[s, q, k, :] = rows[s, q, k, :] * w16[e]
            return carry
        lax.fori_loop(0, QR, qbody, 0)

    for d in issue_in(0, 0):
        d.wait()
    issue_gathers(0)

    ngrp = lax.select(tid == NW - 1, LAST_NGRP, NGRP)

    def group(g, carry):
        j0 = 2 * g
        @pl.when(g > 0)
        def _():
            drain_scatters(1)
        issue_in(j0 + 1, 1)
        drain_gathers(0)
        compute(0)
        issue_scatters(0)
        drain_in(j0 + 1, 1)
        issue_gathers(1)
        drain_scatters(0)
        drain_gathers(1)
        compute(1)
        issue_scatters(1)

        @pl.when(g < ngrp - 1)
        def _():
            issue_in(j0 + 2, 0)
            drain_in(j0 + 2, 0)
            issue_gathers(0)
        return carry

    lax.fori_loop(0, ngrp, group, 0)
    drain_scatters(1)
    plsc.subcore_barrier()

    sl = pl.ds(sid * PT, PT)
    pltpu.sync_copy(acc_sh.at[sl], accp.at[cid, sl])

    @pl.when(cid == 0)
    def _():
        pltpu.sync_copy(c_sh.at[sl], c0.at[sl])

    @pl.when(cid == 1)
    def _():
        pltpu.sync_copy(c_sh.at[sl], c1.at[sl])


NR8 = N2 // 8
BR8 = NR8 // GRID


def _dense1_body(xf_ref, w8_ref, d0_ref, d1_ref, e8_ref, dinv_ref, y_ref):
    i = pl.program_id(0)
    deg = d0_ref[...] + d1_ref[...] + 1.0
    dv = lax.rsqrt(deg)
    node = (lax.broadcasted_iota(jnp.int32, (BR8, 8), 0) + i * BR8) * 8 \
        + lax.broadcasted_iota(jnp.int32, (BR8, 8), 1)
    dv = jnp.where(node < N, dv, 0.0)
    dr = jnp.dot(dv, e8_ref[...], preferred_element_type=jnp.float32)
    xw = jnp.dot(xf_ref[...], w8_ref[...], preferred_element_type=jnp.float32)
    dinv_ref[...] = dv
    y_ref[...] = dr * xw


def _dense1(xf, W8, d0v, d1v, E8):
    return pl.pallas_call(
        _dense1_body,
        grid=(GRID,),
        in_specs=[
            pl.BlockSpec((BR8, 128), lambda i: (i, 0)),
            pl.BlockSpec((128, 128), lambda i: (0, 0)),
            pl.BlockSpec((BR8, 8), lambda i: (i, 0)),
            pl.BlockSpec((BR8, 8), lambda i: (i, 0)),
            pl.BlockSpec((8, 128), lambda i: (0, 0)),
        ],
        out_specs=[
            pl.BlockSpec((BR8, 8), lambda i: (i, 0)),
            pl.BlockSpec((BR8, 128), lambda i: (i, 0)),
        ],
        out_shape=[
            jax.ShapeDtypeStruct((NR8, 8), jnp.float32),
            jax.ShapeDtypeStruct((NR8, 128), jnp.float32),
        ],
    )(xf, W8, d0v, d1v, E8)


def _dense2_body(accp_ref, y_ref, dinv_ref, c0_ref, c1_ref, b1r_ref,
                 e8_ref, w2_ref, b2_ref, wlin_ref, blin_ref, out_ref, s_ref):
    i = pl.program_id(0)
    dv = dinv_ref[...]
    dr = jnp.dot(dv, e8_ref[...], preferred_element_type=jnp.float32)
    acc = accp_ref[0] + accp_ref[1]
    h = jax.nn.relu((acc + y_ref[...]) * dr + b1r_ref[...])
    coef = dv * (c0_ref[...] + c1_ref[...] + dv)
    coefr = jnp.dot(coef, e8_ref[...], preferred_element_type=jnp.float32)
    part = jnp.sum(coefr * h, axis=0, keepdims=True)

    @pl.when(i == 0)
    def _():
        s_ref[...] = part

    @pl.when(i > 0)
    def _():
        s_ref[...] = s_ref[...] + part

    @pl.when(i == GRID - 1)
    def _():
        s128 = s_ref[...]
        s16 = s128[:, 0:16]
        for gg in range(1, 8):
            s16 = s16 + s128[:, gg * 16:(gg + 1) * 16]
        g = jnp.dot(s16 / float(N), w2_ref[...],
                    preferred_element_type=jnp.float32) + b2_ref[...]
        out_ref[...] = jnp.dot(g, wlin_ref[...],
                               preferred_element_type=jnp.float32) \
            + blin_ref[...]


def _dense2(accp8, y8, dinv8, c08, c18, b1r, E8, W2, b2, Wlin, blin):
    return pl.pallas_call(
        _dense2_body,
        grid=(GRID,),
        in_specs=[
            pl.BlockSpec((2, BR8, 128), lambda i: (0, i, 0)),
            pl.BlockSpec((BR8, 128), lambda i: (i, 0)),
            pl.BlockSpec((BR8, 8), lambda i: (i, 0)),
            pl.BlockSpec((BR8, 8), lambda i: (i, 0)),
            pl.BlockSpec((BR8, 8), lambda i: (i, 0)),
            pl.BlockSpec((1, 128), lambda i: (0, 0)),
            pl.BlockSpec((8, 128), lambda i: (0, 0)),
            pl.BlockSpec((F, F), lambda i: (0, 0)),
            pl.BlockSpec((1, F), lambda i: (0, 0)),
            pl.BlockSpec((F, 1), lambda i: (0, 0)),
            pl.BlockSpec((1, 1), lambda i: (0, 0)),
        ],
        out_specs=pl.BlockSpec((1, 1), lambda i: (0, 0)),
        out_shape=jax.ShapeDtypeStruct((1, 1), jnp.float32),
        scratch_shapes=[pltpu.VMEM((1, 128), jnp.float32)],
    )(accp8, y8, dinv8, c08, c18, b1r, E8, W2, b2, Wlin, blin)


def kernel(x, edge_index, edge_attr, W1, b1, W2, b2, Wlin, blin):
    src2 = edge_index[0].reshape(-1, 128)
    dst2 = edge_index[1].reshape(-1, 128)
    w2e = edge_attr.reshape(-1, 128)
    xf = jnp.pad(x, ((0, N2 - N), (0, 0))).reshape(NR8, 128)
    eye8 = jnp.eye(8, dtype=jnp.float32)
    W8 = jnp.kron(eye8, W1)
    E8 = jnp.kron(eye8, jnp.ones((1, F), jnp.float32))
    b1r = jnp.tile(b1, 8).reshape(1, 128)

    d0, d1 = _deg_kernel(dst2, w2e)
    dinv8, y8 = _dense1(xf, W8, d0.reshape(NR8, 8), d1.reshape(NR8, 8), E8)
    accp, c0, c1 = _edge_kernel(src2, dst2, w2e,
                                y8.reshape(N2, F), dinv8.reshape(N2))
    out = _dense2(accp.reshape(2, NR8, 128), y8,
                  dinv8, c0.reshape(NR8, 8), c1.reshape(NR8, 8),
                  b1r, E8, W2, b2.reshape(1, F), Wlin, blin.reshape(1, 1))
    return out
